# baseline (speedup 1.0000x reference)
import functools

import jax
import jax.numpy as jnp
from jax import lax
from jax.experimental import pallas as pl
from jax.experimental.pallas import tpu as pltpu
from jax.experimental.pallas import tpu_sc as plsc


def _sc_dot_lookup(B, D):
    info = plsc.get_sparse_core_info()
    NC, NS, L = info.num_cores, info.num_subcores, info.num_lanes
    NW = NC * NS
    assert B % NW == 0
    b_per_w = B // NW
    n_chunks = 4
    chunk = b_per_w // n_chunks

    mesh = plsc.VectorSubcoreMesh(core_axis_name="c", subcore_axis_name="s")

    @functools.partial(
        pl.kernel,
        mesh=mesh,
        out_type=jax.ShapeDtypeStruct((B,), jnp.float32),
        compiler_params=pltpu.CompilerParams(
            needs_layout_passes=False, use_tc_tiling_on_sc=False),
        scratch_types=[
            pltpu.VMEM((n_chunks, chunk), jnp.int32),
            pltpu.VMEM((n_chunks, chunk), jnp.int32),
            pltpu.VMEM((b_per_w, D), jnp.float32),
            pltpu.VMEM((b_per_w, D), jnp.float32),
            pltpu.VMEM((b_per_w,), jnp.float32),
            pltpu.SemaphoreType.DMA,
        ],
    )
    def k(tgt_hbm, ctx_hbm, ttab_hbm, ctab_hbm, out_hbm,
          idx_t, idx_c, rows_t, rows_c, out_v, sem):
        wid = lax.axis_index("s") * NC + lax.axis_index("c")
        base = wid * b_per_w

        for j in range(n_chunks):
            pltpu.sync_copy(tgt_hbm.at[pl.ds(base + j * chunk, chunk)], idx_t.at[j])
            pltpu.sync_copy(ctx_hbm.at[pl.ds(base + j * chunk, chunk)], idx_c.at[j])

        copies = []
        for j in range(n_chunks):
            copies.append(pltpu.async_copy(
                ttab_hbm.at[idx_t.at[j]], rows_t.at[pl.ds(j * chunk, chunk)], sem))
            copies.append(pltpu.async_copy(
                ctab_hbm.at[idx_c.at[j]], rows_c.at[pl.ds(j * chunk, chunk)], sem))
        for c in copies:
            c.wait()

        lanes = lax.iota(jnp.int32, L)

        def body(g, carry):
            row = g * L + lanes
            acc = jnp.zeros((L,), jnp.float32)
            for d in range(D):
                col = jnp.bitwise_and(lanes + d, D - 1)
                tg = plsc.load_gather(rows_t, [row, col])
                cg = plsc.load_gather(rows_c, [row, col])
                acc = acc + tg * cg
            out_v[pl.ds(g * L, L)] = acc
            return carry

        lax.fori_loop(0, b_per_w // L, body, 0)

        pltpu.sync_copy(out_v, out_hbm.at[pl.ds(base, b_per_w)])

    return k


def kernel(target, context, target_table, context_table):
    B = target.shape[0]
    D = target_table.shape[1]
    k = _sc_dot_lookup(B, D)
    return k(target.astype(jnp.int32), context.astype(jnp.int32),
             target_table, context_table)
